# baseline (speedup 1.0000x reference)
import functools

import jax
import jax.numpy as jnp
from jax import lax
from jax.experimental import pallas as pl
from jax.experimental.pallas import tpu as pltpu
from jax.experimental.pallas import tpu_sc as plsc

B = 16384
D = 64
NC = 2
NS = 16
NW = NC * NS
BPW = B // NW
L = 16
CHUNK = 128
NCHUNK = BPW // CHUNK
GPC = CHUNK // L


def _sc_scores(u_pos, v, u_table2, v_table2):
    mesh = plsc.VectorSubcoreMesh(core_axis_name="c", subcore_axis_name="s")

    @functools.partial(
        pl.kernel,
        out_type=jax.ShapeDtypeStruct((B,), jnp.float32),
        mesh=mesh,
        compiler_params=pltpu.CompilerParams(needs_layout_passes=False),
        scratch_types=[
            pltpu.VMEM((NCHUNK, CHUNK), jnp.int32),
            pltpu.VMEM((NCHUNK, CHUNK), jnp.int32),
            pltpu.VMEM((NCHUNK, CHUNK), jnp.int32),
            pltpu.VMEM((NCHUNK, CHUNK), jnp.int32),
            pltpu.VMEM((CHUNK, 2 * D), jnp.float32),
            pltpu.VMEM((CHUNK, 2 * D), jnp.float32),
            pltpu.VMEM((CHUNK, 2 * D), jnp.float32),
            pltpu.VMEM((CHUNK, 2 * D), jnp.float32),
            pltpu.VMEM((BPW,), jnp.float32),
            pltpu.SemaphoreType.DMA,
            pltpu.SemaphoreType.DMA,
            pltpu.SemaphoreType.DMA,
            pltpu.SemaphoreType.DMA,
        ],
    )
    def k(u_pos_hbm, v_hbm, u_tab_hbm, v_tab_hbm, out_hbm,
          uidx_v, vidx_v, uhalf_v, vhalf_v,
          ubuf0, ubuf1, vbuf0, vbuf1, out_v,
          su0, su1, sv0, sv1):
        wid = lax.axis_index("s") * NC + lax.axis_index("c")
        base = wid * BPW
        ubufs, vbufs = [ubuf0, ubuf1], [vbuf0, vbuf1]
        sus, svs = [su0, su1], [sv0, sv1]

        for c in range(NCHUNK):
            pltpu.sync_copy(u_pos_hbm.at[pl.ds(base + c * CHUNK, CHUNK)],
                            uidx_v.at[c])
            pltpu.sync_copy(v_hbm.at[pl.ds(base + c * CHUNK, CHUNK)],
                            vidx_v.at[c])
            for q in range(CHUNK // L):
                sl = pl.ds(q * L, L)
                uhalf_v[c, sl] = uidx_v[c, sl] >> 1
                vhalf_v[c, sl] = vidx_v[c, sl] >> 1

        def fire(c):
            s = c % 2
            pltpu.async_copy(u_tab_hbm.at[uhalf_v.at[c]], ubufs[s], sus[s])
            pltpu.async_copy(v_tab_hbm.at[vhalf_v.at[c]], vbufs[s], svs[s])

        fire(0)
        lanes = lax.iota(jnp.int32, L)
        for c in range(NCHUNK):
            s = c % 2
            if c + 1 < NCHUNK:
                fire(c + 1)
            pltpu.make_async_copy(u_tab_hbm.at[uhalf_v.at[c]],
                                  ubufs[s], sus[s]).wait()
            pltpu.make_async_copy(v_tab_hbm.at[vhalf_v.at[c]],
                                  vbufs[s], svs[s]).wait()
            ub, vb = ubufs[s], vbufs[s]

            def group(g, carry, c=c, ub=ub, vb=vb):
                rid = g * L + lanes
                sl = pl.dslice(g * L, L)
                ucol = (uidx_v[c, sl] & 1) * D
                vcol = (vidx_v[c, sl] & 1) * D
                acc = jnp.zeros((L,), jnp.float32)
                for j in range(D):
                    uu = plsc.load_gather(ub, [rid, ucol + j])
                    vv = plsc.load_gather(vb, [rid, vcol + j])
                    acc = acc + uu * vv
                out_v[pl.ds(c * CHUNK + g * L, L)] = acc
                return carry

            lax.fori_loop(0, GPC, group, 0)

        pltpu.sync_copy(out_v, out_hbm.at[pl.ds(base, BPW)])

    return k(u_pos, v, u_table2, v_table2)


def _logsigmoid_tc(scores):
    x = scores.reshape(B // 128, 128)

    def body(x_ref, o_ref):
        o_ref[...] = jax.nn.log_sigmoid(x_ref[...])

    y = pl.pallas_call(
        body,
        out_shape=jax.ShapeDtypeStruct((B // 128, 128), jnp.float32),
    )(x)
    return y.reshape(B)


def kernel(u_pos, v, u_table, v_table):
    vocab = u_table.shape[0]
    u2 = u_table.reshape(vocab // 2, 2 * D)
    v2 = v_table.reshape(vocab // 2, 2 * D)
    scores = _sc_scores(u_pos, v, u2, v2)
    return _logsigmoid_tc(scores)


# speedup vs baseline: 3.1213x; 3.1213x over previous
import functools

import jax
import jax.numpy as jnp
from jax import lax
from jax.experimental import pallas as pl
from jax.experimental.pallas import tpu as pltpu
from jax.experimental.pallas import tpu_sc as plsc

B = 16384
D = 64
VOCAB = 1000000
NC = 2
NS = 16
NW = NC * NS
BPW = B // NW
L = 16
CW = 256
NG = B // L
RING = 8
POSB = 14

_params = pltpu.CompilerParams(needs_layout_passes=False)
_mesh = plsc.VectorSubcoreMesh(core_axis_name="c", subcore_axis_name="s")


def _extract_stage(u_pos, v, u_tabT, v_tabT):
    @functools.partial(
        pl.kernel,
        out_type=(jax.ShapeDtypeStruct((B * D,), jnp.float32),
                  jax.ShapeDtypeStruct((B * D,), jnp.float32)),
        mesh=_mesh,
        compiler_params=_params,
        scratch_types=[
            pltpu.VMEM((B,), jnp.int32),
            pltpu.VMEM((B + L,), jnp.int32),
            pltpu.VMEM((2, D, CW), jnp.float32),
            pltpu.VMEM((2 * L,), jnp.int32),
            pltpu.VMEM((RING, D), jnp.float32),
            pltpu.SemaphoreType.DMA,
            pltpu.SemaphoreType.DMA,
            pltpu.SemaphoreType.DMA,
        ],
    )
    def k(u_pos_hbm, v_hbm, u_tab_hbm, v_tab_hbm, ustage_hbm, vstage_hbm,
          idx_v, pk_v, chunk_v, hq_v, ring_v, sem_c0, sem_c1, sem_r):
        wid = lax.axis_index("s") * NC + lax.axis_index("c")
        lo = ((wid * (VOCAB // NW)) // 128) * 128
        hi = jnp.where(wid == NW - 1, VOCAB,
                       (((wid + 1) * (VOCAB // NW)) // 128) * 128)
        lo = pl.multiple_of(lo, 128)
        nchunks = (hi - lo + CW - 1) // CW
        lanes = lax.iota(jnp.int32, L)

        def table_pass(idx_hbm, tab_hbm, stage_hbm):
            pltpu.sync_copy(idx_hbm, idx_v)

            def compress(g, n):
                r = idx_v[pl.ds(g * L, L)]
                m = (r >= lo) & (r < hi)
                packed = ((r - lo) << POSB) | (g * L + lanes)
                plsc.store_compressed(pk_v.at[pl.ds(n, L)], packed, mask=m)
                cnt = plsc.all_reduce_population_count(m)[0]
                return n + cnt

            n_mine = lax.fori_loop(0, NG, compress, 0)

            def chunk_start(kk):
                s = jnp.where(kk == nchunks - 1, hi - CW, lo + kk * CW)
                return pl.multiple_of(s, 128)

            sems = [sem_c0, sem_c1]

            def fire(kk, slot):
                pltpu.async_copy(
                    tab_hbm.at[:, pl.ds(chunk_start(kk), CW)],
                    chunk_v.at[slot], sems[slot])

            def process(kk, slot, dma_in):
                start = chunk_start(kk)
                own_lo = lo + kk * CW
                own_hi = jnp.minimum(lo + (kk + 1) * CW, hi)
                pltpu.make_async_copy(
                    tab_hbm.at[:, pl.ds(start, CW)],
                    chunk_v.at[slot], sems[slot]).wait()
                cbuf = chunk_v.at[slot]

                def per_group(g, dma_cnt):
                    p = pk_v[pl.ds(g * L, L)]
                    r = (p >> POSB) + lo
                    m = (lanes < (n_mine - g * L)) & (r >= own_lo) & (r < own_hi)
                    plsc.store_compressed(hq_v.at[pl.ds(0, L)], p, mask=m)
                    nhit = plsc.all_reduce_population_count(m)[0]

                    def per_hit(e, dc):
                        pe = hq_v[pl.ds(e, L)][0]
                        j = (pe >> POSB) + lo - start
                        pos = pe & ((1 << POSB) - 1)
                        slot_r = lax.rem(dc, RING)

                        @pl.when((slot_r == 0) & (dc > 0))
                        def _():
                            for _ in range(RING):
                                pltpu.make_async_copy(
                                    ring_v.at[0],
                                    stage_hbm.at[pl.ds(0, D)], sem_r).wait()

                        jv = lanes * 0 + j
                        for f in range(D // L):
                            ring_v[slot_r, pl.ds(f * L, L)] = (
                                plsc.load_gather(cbuf, [f * L + lanes, jv]))
                        pltpu.async_copy(
                            ring_v.at[slot_r],
                            stage_hbm.at[pl.ds(pos * D, D)], sem_r)
                        return dc + 1

                    return lax.fori_loop(0, nhit, per_hit, dma_cnt)

                ngrp = (n_mine + L - 1) // L
                return lax.fori_loop(0, ngrp, per_group, dma_in)

            fire(0, 0)

            def per_pair(p, carry):
                k0 = 2 * p
                k1 = k0 + 1

                @pl.when(k1 < nchunks)
                def _():
                    fire(k1, 1)

                carry = process(k0, 0, carry)

                @pl.when(k0 + 2 < nchunks)
                def _():
                    fire(k0 + 2, 0)

                return lax.cond(k1 < nchunks,
                                lambda c: process(k1, 1, c),
                                lambda c: c, carry)

            npairs = (nchunks + 1) // 2
            total_dma = lax.fori_loop(0, npairs, per_pair, 0)

            rem = jnp.where(
                total_dma > 0,
                total_dma - ((total_dma - 1) // RING) * RING, 0)

            def drain(e, carry):
                @pl.when(e < rem)
                def _():
                    pltpu.make_async_copy(
                        ring_v.at[0], stage_hbm.at[pl.ds(0, D)], sem_r).wait()
                return carry

            lax.fori_loop(0, RING, drain, 0)

        table_pass(u_pos_hbm, u_tab_hbm, ustage_hbm)
        table_pass(v_hbm, v_tab_hbm, vstage_hbm)

    return k(u_pos, v, u_tabT, v_tabT)


def _dot_stage(ustage, vstage):
    @functools.partial(
        pl.kernel,
        out_type=jax.ShapeDtypeStruct((B,), jnp.float32),
        mesh=_mesh,
        compiler_params=_params,
        scratch_types=[
            pltpu.VMEM((BPW * D,), jnp.float32),
            pltpu.VMEM((BPW * D,), jnp.float32),
            pltpu.VMEM((BPW,), jnp.float32),
            pltpu.VMEM((L * L,), jnp.float32),
            pltpu.SemaphoreType.DMA,
            pltpu.SemaphoreType.DMA,
        ],
    )
    def k(ustage_hbm, vstage_hbm, out_hbm, urows_v, vrows_v, out_v, tbuf_v,
          sem_u, sem_v):
        wid = lax.axis_index("s") * NC + lax.axis_index("c")
        base = wid * BPW
        cu = pltpu.async_copy(
            ustage_hbm.at[pl.ds(base * D, BPW * D)], urows_v, sem_u)
        cv = pltpu.async_copy(
            vstage_hbm.at[pl.ds(base * D, BPW * D)], vrows_v, sem_v)
        cu.wait()
        cv.wait()
        lanes = lax.iota(jnp.int32, L)

        def group(g, carry):
            base_r = g * L
            for r in range(L):
                s = jnp.zeros((L,), jnp.float32)
                for j in range(D // L):
                    uu = urows_v[pl.ds((base_r + r) * D + j * L, L)]
                    vv = vrows_v[pl.ds((base_r + r) * D + j * L, L)]
                    s = s + uu * vv
                plsc.store_scatter(tbuf_v, [lanes * L + r], s)
            acc = jnp.zeros((L,), jnp.float32)
            for kk in range(L):
                acc = acc + tbuf_v[pl.ds(kk * L, L)]
            out_v[pl.ds(g * L, L)] = acc
            return carry

        lax.fori_loop(0, BPW // L, group, 0)
        pltpu.sync_copy(out_v, out_hbm.at[pl.ds(base, BPW)])

    return k(ustage, vstage)


def _logsigmoid_tc(scores):
    x = scores.reshape(B // 128, 128)

    def body(x_ref, o_ref):
        o_ref[...] = jax.nn.log_sigmoid(x_ref[...])

    y = pl.pallas_call(
        body,
        out_shape=jax.ShapeDtypeStruct((B // 128, 128), jnp.float32),
    )(x)
    return y.reshape(B)


def kernel(u_pos, v, u_table, v_table):
    ustage, vstage = _extract_stage(u_pos, v, u_table.T, v_table.T)
    scores = _dot_stage(ustage, vstage)
    return _logsigmoid_tc(scores)


# speedup vs baseline: 4.1511x; 1.3299x over previous
import functools

import jax
import jax.numpy as jnp
from jax import lax
from jax.experimental import pallas as pl
from jax.experimental.pallas import tpu as pltpu
from jax.experimental.pallas import tpu_sc as plsc

B = 16384
D = 64
VOCAB = 1000000
NC = 2
NS = 16
NW = NC * NS
BPW = B // NW
L = 16
CW = 512
NG = B // L
RING = 8
POSB = 14

_params = pltpu.CompilerParams(needs_layout_passes=False)
_mesh = plsc.VectorSubcoreMesh(core_axis_name="c", subcore_axis_name="s")


def _extract_stage(u_pos, v, u_tabT, v_tabT):
    @functools.partial(
        pl.kernel,
        out_type=(jax.ShapeDtypeStruct((B * D,), jnp.float32),
                  jax.ShapeDtypeStruct((B * D,), jnp.float32)),
        mesh=_mesh,
        compiler_params=_params,
        scratch_types=[
            pltpu.VMEM((B,), jnp.int32),
            pltpu.VMEM((B + L,), jnp.int32),
            pltpu.VMEM((2, D, CW), jnp.float32),
            pltpu.VMEM((2 * L,), jnp.int32),
            pltpu.VMEM((RING, D), jnp.float32),
            pltpu.SemaphoreType.DMA,
            pltpu.SemaphoreType.DMA,
            pltpu.SemaphoreType.DMA,
        ],
    )
    def k(u_pos_hbm, v_hbm, u_tab_hbm, v_tab_hbm, ustage_hbm, vstage_hbm,
          idx_v, pk_v, chunk_v, hq_v, ring_v, sem_c0, sem_c1, sem_r):
        wid = lax.axis_index("s") * NC + lax.axis_index("c")
        lo = ((wid * (VOCAB // NW)) // 128) * 128
        hi = jnp.where(wid == NW - 1, VOCAB,
                       (((wid + 1) * (VOCAB // NW)) // 128) * 128)
        lo = pl.multiple_of(lo, 128)
        nchunks = (hi - lo + CW - 1) // CW
        lanes = lax.iota(jnp.int32, L)

        def table_pass(idx_hbm, tab_hbm, stage_hbm):
            pltpu.sync_copy(idx_hbm, idx_v)

            def compress(g, n):
                r = idx_v[pl.ds(g * L, L)]
                m = (r >= lo) & (r < hi)
                packed = ((r - lo) << POSB) | (g * L + lanes)
                plsc.store_compressed(pk_v.at[pl.ds(n, L)], packed, mask=m)
                cnt = plsc.all_reduce_population_count(m)[0]
                return n + cnt

            n_mine = lax.fori_loop(0, NG, compress, 0)

            def chunk_start(kk):
                s = jnp.where(kk == nchunks - 1, hi - CW, lo + kk * CW)
                return pl.multiple_of(s, 128)

            sems = [sem_c0, sem_c1]

            def fire(kk, slot):
                pltpu.async_copy(
                    tab_hbm.at[:, pl.ds(chunk_start(kk), CW)],
                    chunk_v.at[slot], sems[slot])

            def process(kk, slot, dma_in):
                start = chunk_start(kk)
                own_lo = lo + kk * CW
                own_hi = jnp.minimum(lo + (kk + 1) * CW, hi)
                pltpu.make_async_copy(
                    tab_hbm.at[:, pl.ds(start, CW)],
                    chunk_v.at[slot], sems[slot]).wait()
                cbuf = chunk_v.at[slot]

                def per_group(g, dma_cnt):
                    p = pk_v[pl.ds(g * L, L)]
                    r = (p >> POSB) + lo
                    m = (lanes < (n_mine - g * L)) & (r >= own_lo) & (r < own_hi)
                    plsc.store_compressed(hq_v.at[pl.ds(0, L)], p, mask=m)
                    nhit = plsc.all_reduce_population_count(m)[0]

                    def per_hit(e, dc):
                        pe = hq_v[pl.ds(e, L)][0]
                        j = (pe >> POSB) + lo - start
                        pos = pe & ((1 << POSB) - 1)
                        slot_r = lax.rem(dc, RING)

                        @pl.when((slot_r == 0) & (dc > 0))
                        def _():
                            for _ in range(RING):
                                pltpu.make_async_copy(
                                    ring_v.at[0],
                                    stage_hbm.at[pl.ds(0, D)], sem_r).wait()

                        jv = lanes * 0 + j
                        for f in range(D // L):
                            ring_v[slot_r, pl.ds(f * L, L)] = (
                                plsc.load_gather(cbuf, [f * L + lanes, jv]))
                        pltpu.async_copy(
                            ring_v.at[slot_r],
                            stage_hbm.at[pl.ds(pos * D, D)], sem_r)
                        return dc + 1

                    return lax.fori_loop(0, nhit, per_hit, dma_cnt)

                ngrp = (n_mine + L - 1) // L
                return lax.fori_loop(0, ngrp, per_group, dma_in)

            fire(0, 0)

            def per_pair(p, carry):
                k0 = 2 * p
                k1 = k0 + 1

                @pl.when(k1 < nchunks)
                def _():
                    fire(k1, 1)

                carry = process(k0, 0, carry)

                @pl.when(k0 + 2 < nchunks)
                def _():
                    fire(k0 + 2, 0)

                return lax.cond(k1 < nchunks,
                                lambda c: process(k1, 1, c),
                                lambda c: c, carry)

            npairs = (nchunks + 1) // 2
            total_dma = lax.fori_loop(0, npairs, per_pair, 0)

            rem = jnp.where(
                total_dma > 0,
                total_dma - ((total_dma - 1) // RING) * RING, 0)

            def drain(e, carry):
                @pl.when(e < rem)
                def _():
                    pltpu.make_async_copy(
                        ring_v.at[0], stage_hbm.at[pl.ds(0, D)], sem_r).wait()
                return carry

            lax.fori_loop(0, RING, drain, 0)

        table_pass(u_pos_hbm, u_tab_hbm, ustage_hbm)
        table_pass(v_hbm, v_tab_hbm, vstage_hbm)

    return k(u_pos, v, u_tabT, v_tabT)


def _dot_stage(ustage, vstage):
    @functools.partial(
        pl.kernel,
        out_type=jax.ShapeDtypeStruct((B,), jnp.float32),
        mesh=_mesh,
        compiler_params=_params,
        scratch_types=[
            pltpu.VMEM((BPW * D,), jnp.float32),
            pltpu.VMEM((BPW * D,), jnp.float32),
            pltpu.VMEM((BPW,), jnp.float32),
            pltpu.VMEM((L * L,), jnp.float32),
            pltpu.SemaphoreType.DMA,
            pltpu.SemaphoreType.DMA,
        ],
    )
    def k(ustage_hbm, vstage_hbm, out_hbm, urows_v, vrows_v, out_v, tbuf_v,
          sem_u, sem_v):
        wid = lax.axis_index("s") * NC + lax.axis_index("c")
        base = wid * BPW
        cu = pltpu.async_copy(
            ustage_hbm.at[pl.ds(base * D, BPW * D)], urows_v, sem_u)
        cv = pltpu.async_copy(
            vstage_hbm.at[pl.ds(base * D, BPW * D)], vrows_v, sem_v)
        cu.wait()
        cv.wait()
        lanes = lax.iota(jnp.int32, L)

        def group(g, carry):
            base_r = g * L
            for r in range(L):
                s = jnp.zeros((L,), jnp.float32)
                for j in range(D // L):
                    uu = urows_v[pl.ds((base_r + r) * D + j * L, L)]
                    vv = vrows_v[pl.ds((base_r + r) * D + j * L, L)]
                    s = s + uu * vv
                plsc.store_scatter(tbuf_v, [lanes * L + r], s)
            acc = jnp.zeros((L,), jnp.float32)
            for kk in range(L):
                acc = acc + tbuf_v[pl.ds(kk * L, L)]
            out_v[pl.ds(g * L, L)] = acc
            return carry

        lax.fori_loop(0, BPW // L, group, 0)
        pltpu.sync_copy(out_v, out_hbm.at[pl.ds(base, BPW)])

    return k(ustage, vstage)


def _logsigmoid_tc(scores):
    x = scores.reshape(B // 128, 128)

    def body(x_ref, o_ref):
        o_ref[...] = jax.nn.log_sigmoid(x_ref[...])

    y = pl.pallas_call(
        body,
        out_shape=jax.ShapeDtypeStruct((B // 128, 128), jnp.float32),
    )(x)
    return y.reshape(B)


def kernel(u_pos, v, u_table, v_table):
    ustage, vstage = _extract_stage(u_pos, v, u_table.T, v_table.T)
    scores = _dot_stage(ustage, vstage)
    return _logsigmoid_tc(scores)


# speedup vs baseline: 4.5990x; 1.1079x over previous
import functools

import jax
import jax.numpy as jnp
from jax import lax
from jax.experimental import pallas as pl
from jax.experimental.pallas import tpu as pltpu
from jax.experimental.pallas import tpu_sc as plsc

B = 16384
D = 64
VOCAB = 1000000
NC = 2
NS = 16
NW = NC * NS
BPW = B // NW
L = 16
CW = 512
NG = B // L
RING = 8
POSB = 14
SL = 4096

_params = pltpu.CompilerParams(needs_layout_passes=False)
_mesh = plsc.VectorSubcoreMesh(core_axis_name="c", subcore_axis_name="s")


def _extract_stage(u_pos, v, u_tabT, v_tabT):
    @functools.partial(
        pl.kernel,
        out_type=(jax.ShapeDtypeStruct((B * D,), jnp.float32),
                  jax.ShapeDtypeStruct((B * D,), jnp.float32)),
        mesh=_mesh,
        compiler_params=_params,
        scratch_types=[
            pltpu.VMEM((SL,), jnp.int32),
            pltpu.VMEM((B + L,), jnp.int32),
            pltpu.VMEM((3, D, CW), jnp.float32),
            pltpu.VMEM((2 * L,), jnp.int32),
            pltpu.VMEM((RING, D), jnp.float32),
            pltpu.SemaphoreType.DMA,
            pltpu.SemaphoreType.DMA,
            pltpu.SemaphoreType.DMA,
            pltpu.SemaphoreType.DMA,
        ],
    )
    def k(u_pos_hbm, v_hbm, u_tab_hbm, v_tab_hbm, ustage_hbm, vstage_hbm,
          idx_v, pk_v, chunk_v, hq_v, ring_v, sem_c0, sem_c1, sem_c2, sem_r):
        wid = lax.axis_index("s") * NC + lax.axis_index("c")
        lo = ((wid * (VOCAB // NW)) // 128) * 128
        hi = jnp.where(wid == NW - 1, VOCAB,
                       (((wid + 1) * (VOCAB // NW)) // 128) * 128)
        lo = pl.multiple_of(lo, 128)
        nchunks = (hi - lo + CW - 1) // CW
        lanes = lax.iota(jnp.int32, L)

        def table_pass(idx_hbm, tab_hbm, stage_hbm):
            def section(sec, n):
                pltpu.sync_copy(idx_hbm.at[pl.ds(sec * SL, SL)], idx_v)

                def compress(g, nn):
                    r = idx_v[pl.ds(g * L, L)]
                    m = (r >= lo) & (r < hi)
                    packed = ((r - lo) << POSB) | (sec * SL + g * L + lanes)
                    plsc.store_compressed(pk_v.at[pl.ds(nn, L)], packed,
                                          mask=m)
                    cnt = plsc.all_reduce_population_count(m)[0]
                    return nn + cnt

                return lax.fori_loop(0, SL // L, compress, n)

            n_mine = lax.fori_loop(0, B // SL, section, 0)

            def chunk_start(kk):
                s = jnp.where(kk == nchunks - 1, hi - CW, lo + kk * CW)
                return pl.multiple_of(s, 128)

            sems = [sem_c0, sem_c1, sem_c2]

            def fire(kk, slot):
                pltpu.async_copy(
                    tab_hbm.at[:, pl.ds(chunk_start(kk), CW)],
                    chunk_v.at[slot], sems[slot])

            def process(kk, slot, dma_in):
                start = chunk_start(kk)
                own_lo = lo + kk * CW
                own_hi = jnp.minimum(lo + (kk + 1) * CW, hi)
                pltpu.make_async_copy(
                    tab_hbm.at[:, pl.ds(start, CW)],
                    chunk_v.at[slot], sems[slot]).wait()
                cbuf = chunk_v.at[slot]

                def per_group(g, dma_cnt):
                    p = pk_v[pl.ds(g * L, L)]
                    r = (p >> POSB) + lo
                    m = (lanes < (n_mine - g * L)) & (r >= own_lo) & (r < own_hi)
                    plsc.store_compressed(hq_v.at[pl.ds(0, L)], p, mask=m)
                    nhit = plsc.all_reduce_population_count(m)[0]

                    def per_hit(e, dc):
                        pe = hq_v[pl.ds(e, L)][0]
                        j = (pe >> POSB) + lo - start
                        pos = pe & ((1 << POSB) - 1)
                        slot_r = lax.rem(dc, RING)

                        @pl.when((slot_r == 0) & (dc > 0))
                        def _():
                            for _ in range(RING):
                                pltpu.make_async_copy(
                                    ring_v.at[0],
                                    stage_hbm.at[pl.ds(0, D)], sem_r).wait()

                        jv = lanes * 0 + j
                        for f in range(D // L):
                            ring_v[slot_r, pl.ds(f * L, L)] = (
                                plsc.load_gather(cbuf, [f * L + lanes, jv]))
                        pltpu.async_copy(
                            ring_v.at[slot_r],
                            stage_hbm.at[pl.ds(pos * D, D)], sem_r)
                        return dc + 1

                    return lax.fori_loop(0, nhit, per_hit, dma_cnt)

                ngrp = (n_mine + L - 1) // L
                return lax.fori_loop(0, ngrp, per_group, dma_in)

            fire(0, 0)
            fire(1, 1)

            def per_triple(p, carry):
                for o in range(3):
                    kk = 3 * p + o

                    @pl.when(kk + 2 < nchunks)
                    def _(kk=kk, o=o):
                        fire(kk + 2, (o + 2) % 3)

                    carry = lax.cond(
                        kk < nchunks,
                        lambda c, kk=kk, o=o: process(kk, o, c),
                        lambda c: c, carry)
                return carry

            ntrip = (nchunks + 2) // 3
            total_dma = lax.fori_loop(0, ntrip, per_triple, 0)

            rem = jnp.where(
                total_dma > 0,
                total_dma - ((total_dma - 1) // RING) * RING, 0)

            def drain(e, carry):
                @pl.when(e < rem)
                def _():
                    pltpu.make_async_copy(
                        ring_v.at[0], stage_hbm.at[pl.ds(0, D)], sem_r).wait()
                return carry

            lax.fori_loop(0, RING, drain, 0)

        table_pass(u_pos_hbm, u_tab_hbm, ustage_hbm)
        table_pass(v_hbm, v_tab_hbm, vstage_hbm)

    return k(u_pos, v, u_tabT, v_tabT)


def _dot_stage(ustage, vstage):
    @functools.partial(
        pl.kernel,
        out_type=jax.ShapeDtypeStruct((B,), jnp.float32),
        mesh=_mesh,
        compiler_params=_params,
        scratch_types=[
            pltpu.VMEM((BPW * D,), jnp.float32),
            pltpu.VMEM((BPW * D,), jnp.float32),
            pltpu.VMEM((BPW,), jnp.float32),
            pltpu.VMEM((L * L,), jnp.float32),
            pltpu.SemaphoreType.DMA,
            pltpu.SemaphoreType.DMA,
        ],
    )
    def k(ustage_hbm, vstage_hbm, out_hbm, urows_v, vrows_v, out_v, tbuf_v,
          sem_u, sem_v):
        wid = lax.axis_index("s") * NC + lax.axis_index("c")
        base = wid * BPW
        cu = pltpu.async_copy(
            ustage_hbm.at[pl.ds(base * D, BPW * D)], urows_v, sem_u)
        cv = pltpu.async_copy(
            vstage_hbm.at[pl.ds(base * D, BPW * D)], vrows_v, sem_v)
        cu.wait()
        cv.wait()
        lanes = lax.iota(jnp.int32, L)

        def group(g, carry):
            base_r = g * L
            for r in range(L):
                s = jnp.zeros((L,), jnp.float32)
                for j in range(D // L):
                    uu = urows_v[pl.ds((base_r + r) * D + j * L, L)]
                    vv = vrows_v[pl.ds((base_r + r) * D + j * L, L)]
                    s = s + uu * vv
                plsc.store_scatter(tbuf_v, [lanes * L + r], s)
            acc = jnp.zeros((L,), jnp.float32)
            for kk in range(L):
                acc = acc + tbuf_v[pl.ds(kk * L, L)]
            out_v[pl.ds(g * L, L)] = acc
            return carry

        lax.fori_loop(0, BPW // L, group, 0)
        pltpu.sync_copy(out_v, out_hbm.at[pl.ds(base, BPW)])

    return k(ustage, vstage)


def _logsigmoid_tc(scores):
    x = scores.reshape(B // 128, 128)

    def body(x_ref, o_ref):
        o_ref[...] = jax.nn.log_sigmoid(x_ref[...])

    y = pl.pallas_call(
        body,
        out_shape=jax.ShapeDtypeStruct((B // 128, 128), jnp.float32),
    )(x)
    return y.reshape(B)


def kernel(u_pos, v, u_table, v_table):
    ustage, vstage = _extract_stage(u_pos, v, u_table.T, v_table.T)
    scores = _dot_stage(ustage, vstage)
    return _logsigmoid_tc(scores)
